# baseline (speedup 1.0000x reference)
import functools

import jax
import jax.numpy as jnp
from jax import lax
from jax.experimental import pallas as pl
from jax.experimental.pallas import tpu as pltpu
from jax.experimental.pallas import tpu_sc as plsc

N0 = 10000
N1 = 10000
NNZ = 320000
D = 128

NC = 2
NS = 16
NW = NC * NS
E_PER_W = NNZ // NW
CHUNK = 128
EP = 10240
NCH = EP // CHUNK
ZR = 40
NZC = N1 // ZR
WB = 80
NWC = N1 // WB


def _sc_segment_sum(x0, meta):
    mesh = plsc.VectorSubcoreMesh(core_axis_name="c", subcore_axis_name="s")

    @functools.partial(
        pl.kernel,
        out_type=jax.ShapeDtypeStruct((NC, N1, D), jnp.float32),
        mesh=mesh,
        compiler_params=pltpu.CompilerParams(needs_layout_passes=False),
        scratch_types=[
            pltpu.VMEM((4, 3, CHUNK), jnp.int32),
            pltpu.VMEM((2, CHUNK, D), jnp.float32),
            pltpu.VMEM((ZR, D), jnp.float32),
            pltpu.VMEM((WB, D), jnp.float32),
            pltpu.VMEM_SHARED((N1, D), jnp.float32),
            pltpu.SemaphoreType.DMA((4,)),
            pltpu.SemaphoreType.DMA((2,)),
            pltpu.SemaphoreType.DMA,
        ],
    )
    def k(x0_hbm, meta_hbm, out_hbm,
          meta_v, gath_v, zero_v, copy_v, agg_sh, msem, gsem, zsem):
        cid = lax.axis_index("c")
        s = lax.axis_index("s")
        w = s * NC + cid
        cbase = w * NCH

        with jax.named_scope("sc_zero"):
            zvec = jnp.zeros((16,), jnp.float32)

            def zero_buf(i, carry):
                for j in range(D // 16):
                    zero_v[i, pl.ds(j * 16, 16)] = zvec
                return carry

            lax.fori_loop(0, ZR, zero_buf, 0)

            for k_ in range(15):
                pltpu.async_copy(
                    zero_v, agg_sh.at[pl.ds((k_ * NS + s) * ZR, ZR)], zsem)

            @pl.when(NS * 15 + s < NZC)
            def _():
                pltpu.sync_copy(
                    zero_v, agg_sh.at[pl.ds((NS * 15 + s) * ZR, ZR)])

            for k_ in range(15):
                pltpu.make_async_copy(
                    zero_v, agg_sh.at[pl.ds(0, ZR)], zsem).wait()
            plsc.subcore_barrier()

        def fire_meta(ci, mb):
            pltpu.async_copy(meta_hbm.at[cbase + ci], meta_v.at[mb],
                             msem.at[mb])

        def wait_meta(mb):
            pltpu.make_async_copy(meta_hbm.at[0], meta_v.at[mb],
                                  msem.at[mb]).wait()

        def fire_gather(b, mb):
            pltpu.async_copy(x0_hbm.at[meta_v.at[mb, 0]], gath_v.at[b],
                             gsem.at[b])

        def wait_gather(b):
            pltpu.make_async_copy(x0_hbm.at[pl.ds(0, CHUNK)], gath_v.at[b],
                                  gsem.at[b]).wait()

        scope_main = jax.named_scope("sc_mainloop")
        scope_main.__enter__()
        for k_ in range(4):
            fire_meta(k_, k_)
        wait_meta(0)
        fire_gather(0, 0)

        def turn(c, b, mb):
            wait_gather(b)

            @pl.when(c + 1 < NCH)
            def _():
                wait_meta((mb + 1) % 4)
                fire_gather(b ^ 1, (mb + 1) % 4)

            gref = gath_v.at[b]
            mref = meta_v.at[mb]

            @plsc.parallel_loop(0, CHUNK, 1, unroll=16)
            def _(e):
                vi = plsc.load_gather(
                    mref,
                    [jnp.full((16,), 2, jnp.int32),
                     jnp.full((16,), e, jnp.int32)])
                v16 = plsc.bitcast(vi, jnp.float32)
                for j in range(D // 16):
                    sl = pl.ds(j * 16, 16)
                    gref[e, sl] = gref[e, sl] * v16

            pltpu.sync_copy(gref, agg_sh.at[meta_v.at[mb, 1]], add=True)

            @pl.when(c + 4 < NCH)
            def _():
                fire_meta(c + 4, mb)

        def body(i, carry):
            for k_ in range(4):
                turn(i * 4 + k_, k_ % 2, k_)
            return carry

        lax.fori_loop(0, NCH // 4, body, 0)
        plsc.subcore_barrier()
        scope_main.__exit__(None, None, None)

        with jax.named_scope("sc_writeback"):
            def wb_chunk(idx):
                off = idx * WB
                pltpu.sync_copy(agg_sh.at[pl.ds(off, WB)], copy_v)
                pltpu.sync_copy(copy_v, out_hbm.at[cid, pl.ds(off, WB)])

            for k_ in range(7):
                wb_chunk(k_ * NS + s)

            @pl.when(NS * 7 + s < NWC)
            def _():
                wb_chunk(NS * 7 + s)

    return k(x0, meta)


def _tc_body(p0_ref, p1_ref, w_ref, o_ref):
    acc = p0_ref[...] + p1_ref[...]
    y = jnp.dot(acc, w_ref[...], preferred_element_type=jnp.float32)
    o_ref[...] = jnp.where(y > 0, y, jnp.exp(y) - 1.0)


def _tc_finish(p0, p1, W):
    blk = 400
    grid = N1 // blk
    return pl.pallas_call(
        _tc_body,
        grid=(grid,),
        in_specs=[
            pl.BlockSpec((blk, D), lambda i: (i, 0)),
            pl.BlockSpec((blk, D), lambda i: (i, 0)),
            pl.BlockSpec((D, D), lambda i: (0, 0)),
        ],
        out_specs=pl.BlockSpec((blk, D), lambda i: (i, 0)),
        out_shape=jax.ShapeDtypeStruct((N1, D), jnp.float32),
    )(p0, p1, W)


def kernel(x_0, x_1, nb_indices, nb_values, W):
    rows = nb_indices[0].astype(jnp.int32)
    cols = nb_indices[1].astype(jnp.int32)
    vals_i = lax.bitcast_convert_type(nb_values, jnp.int32)
    m = jnp.stack([cols, rows, vals_i])
    m = m.reshape(3, NW, E_PER_W)
    m = jnp.pad(m, ((0, 0), (0, 0), (0, EP - E_PER_W)))
    m = (m.reshape(3, NW, NCH, CHUNK)
          .transpose(1, 2, 0, 3)
          .reshape(NW * NCH, 3, CHUNK))
    partials = _sc_segment_sum(x_0, m)
    return _tc_finish(partials[0], partials[1], W)


# speedup vs baseline: 2.3999x; 2.3999x over previous
import functools

import jax
import jax.numpy as jnp
from jax import lax
from jax.experimental import pallas as pl
from jax.experimental.pallas import tpu as pltpu
from jax.experimental.pallas import tpu_sc as plsc

N0 = 10000
N1 = 10000
NNZ = 320000
D = 128

NC = 2
NS = 16
NW = NC * NS
E_PER_W = NNZ // NW
CHUNK = 80
NCH = E_PER_W // CHUNK
G = 4
M = 8
ZR = 40
NZC = N1 // ZR
WB = 80
NWC = N1 // WB


def _sc_segment_sum(x0, meta):
    mesh = plsc.VectorSubcoreMesh(core_axis_name="c", subcore_axis_name="s")

    @functools.partial(
        pl.kernel,
        out_type=jax.ShapeDtypeStruct((NC, N1, D), jnp.float32),
        mesh=mesh,
        compiler_params=pltpu.CompilerParams(needs_layout_passes=False),
        scratch_types=[
            pltpu.VMEM((M, 3, CHUNK), jnp.int32),
            pltpu.VMEM((G, CHUNK, D), jnp.float32),
            pltpu.VMEM_SHARED((N1, D), jnp.float32),
            pltpu.SemaphoreType.DMA((M,)),
            pltpu.SemaphoreType.DMA((G,)),
            pltpu.SemaphoreType.DMA((G,)),
            pltpu.SemaphoreType.DMA,
        ],
    )
    def k(x0_hbm, meta_hbm, out_hbm,
          meta_v, gath_v, agg_sh, msem, gsem, ssem, zsem):
        cid = lax.axis_index("c")
        s = lax.axis_index("s")
        w = s * NC + cid
        cbase = w * NCH

        with jax.named_scope("sc_zero"):
            zvec = jnp.zeros((16,), jnp.float32)

            def zero_buf(i, carry):
                for j in range(D // 16):
                    gath_v[0, i, pl.ds(j * 16, 16)] = zvec
                return carry

            lax.fori_loop(0, ZR, zero_buf, 0)
            zero_v = gath_v.at[0, pl.ds(0, ZR)]

            for k_ in range(15):
                pltpu.async_copy(
                    zero_v, agg_sh.at[pl.ds((k_ * NS + s) * ZR, ZR)], zsem)

            @pl.when(NS * 15 + s < NZC)
            def _():
                pltpu.sync_copy(
                    zero_v, agg_sh.at[pl.ds((NS * 15 + s) * ZR, ZR)])

            for k_ in range(15):
                pltpu.make_async_copy(
                    zero_v, agg_sh.at[pl.ds(0, ZR)], zsem).wait()
            plsc.subcore_barrier()

        def fire_meta(ci, mb):
            pltpu.async_copy(meta_hbm.at[cbase + ci], meta_v.at[mb],
                             msem.at[mb])

        def wait_meta(mb):
            pltpu.make_async_copy(meta_hbm.at[0], meta_v.at[mb],
                                  msem.at[mb]).wait()

        def fire_gather(b, mb):
            pltpu.async_copy(x0_hbm.at[meta_v.at[mb, 0]], gath_v.at[b],
                             gsem.at[b])

        def wait_gather(b):
            pltpu.make_async_copy(x0_hbm.at[pl.ds(0, CHUNK)], gath_v.at[b],
                                  gsem.at[b]).wait()

        def wait_scatter(b):
            pltpu.make_async_copy(gath_v.at[b],
                                  agg_sh.at[pl.ds(0, CHUNK)],
                                  ssem.at[b]).wait()

        scope_main = jax.named_scope("sc_mainloop")
        scope_main.__enter__()
        for k_ in range(6):
            fire_meta(k_, k_)
        wait_meta(0)
        fire_gather(0, 0)
        wait_meta(1)
        fire_gather(1, 1)

        def turn(c, carry):
            b = lax.rem(c, G)
            mb = lax.rem(c, M)
            wait_gather(b)

            gref = gath_v.at[b]
            mref = meta_v.at[mb]

            @plsc.parallel_loop(0, CHUNK, 1, unroll=16)
            def _(e):
                vi = plsc.load_gather(
                    mref,
                    [jnp.full((16,), 2, jnp.int32),
                     jnp.full((16,), e, jnp.int32)])
                v16 = plsc.bitcast(vi, jnp.float32)
                for j in range(D // 16):
                    sl = pl.ds(j * 16, 16)
                    gref[e, sl] = gref[e, sl] * v16

            pltpu.async_copy(gref, agg_sh.at[meta_v.at[mb, 1]], ssem.at[b],
                             add=True)

            @pl.when(c >= 2)
            def _():
                wait_scatter(lax.rem(c + 2, G))

            @pl.when(c + 6 < NCH)
            def _():
                fire_meta(c + 6, lax.rem(c + 6, M))

            @pl.when(c + 2 < NCH)
            def _():
                wait_meta(lax.rem(c + 2, M))
                fire_gather(lax.rem(c + 2, G), lax.rem(c + 2, M))

            return carry

        lax.fori_loop(0, NCH, turn, 0)
        wait_scatter((NCH - 2) % G)
        wait_scatter((NCH - 1) % G)
        plsc.subcore_barrier()
        scope_main.__exit__(None, None, None)

        with jax.named_scope("sc_writeback"):
            def wb_chunk(idx):
                off = idx * WB
                pltpu.sync_copy(agg_sh.at[pl.ds(off, WB)],
                                out_hbm.at[cid, pl.ds(off, WB)])

            for k_ in range(7):
                wb_chunk(k_ * NS + s)

            @pl.when(NS * 7 + s < NWC)
            def _():
                wb_chunk(NS * 7 + s)

    return k(x0, meta)


def _tc_body(p0_ref, p1_ref, w_ref, o_ref):
    acc = p0_ref[...] + p1_ref[...]
    y = jnp.dot(acc, w_ref[...], preferred_element_type=jnp.float32)
    o_ref[...] = jnp.where(y > 0, y, jnp.exp(y) - 1.0)


def _tc_finish(p0, p1, W):
    blk = 400
    grid = N1 // blk
    return pl.pallas_call(
        _tc_body,
        grid=(grid,),
        in_specs=[
            pl.BlockSpec((blk, D), lambda i: (i, 0)),
            pl.BlockSpec((blk, D), lambda i: (i, 0)),
            pl.BlockSpec((D, D), lambda i: (0, 0)),
        ],
        out_specs=pl.BlockSpec((blk, D), lambda i: (i, 0)),
        out_shape=jax.ShapeDtypeStruct((N1, D), jnp.float32),
    )(p0, p1, W)


def kernel(x_0, x_1, nb_indices, nb_values, W):
    rows = nb_indices[0].astype(jnp.int32)
    cols = nb_indices[1].astype(jnp.int32)
    vals_i = lax.bitcast_convert_type(nb_values, jnp.int32)
    m = jnp.stack([cols, rows, vals_i])
    m = (m.reshape(3, NW, NCH, CHUNK)
          .transpose(1, 2, 0, 3)
          .reshape(NW * NCH, 3, CHUNK))
    partials = _sc_segment_sum(x_0, m)
    return _tc_finish(partials[0], partials[1], W)


# speedup vs baseline: 2.8141x; 1.1726x over previous
import functools

import jax
import jax.numpy as jnp
from jax import lax
from jax.experimental import pallas as pl
from jax.experimental.pallas import tpu as pltpu
from jax.experimental.pallas import tpu_sc as plsc

N0 = 10000
N1 = 10000
NNZ = 320000
D = 128

NC = 2
NS = 16
NW = NC * NS
E_PER_W = NNZ // NW
CHUNK = 80
NCH = E_PER_W // CHUNK
G = 4
M = 8
ZR = 40
NZC = N1 // ZR
WB = 80
NWC = N1 // WB


def _sc_segment_sum(x0, nbidx, vals):
    mesh = plsc.VectorSubcoreMesh(core_axis_name="c", subcore_axis_name="s")

    @functools.partial(
        pl.kernel,
        out_type=jax.ShapeDtypeStruct((NC, N1, D), jnp.float32),
        mesh=mesh,
        compiler_params=pltpu.CompilerParams(needs_layout_passes=False),
        scratch_types=[
            pltpu.VMEM((M, CHUNK), jnp.int32),
            pltpu.VMEM((M, CHUNK), jnp.int32),
            pltpu.VMEM((M, CHUNK), jnp.float32),
            pltpu.VMEM((G, CHUNK, D), jnp.float32),
            pltpu.VMEM_SHARED((N1, D), jnp.float32),
            pltpu.SemaphoreType.DMA((M,)),
            pltpu.SemaphoreType.DMA((G,)),
            pltpu.SemaphoreType.DMA((G,)),
            pltpu.SemaphoreType.DMA,
        ],
    )
    def k(x0_hbm, nbidx_hbm, vals_hbm, out_hbm,
          cidx_v, ridx_v, vals_v, gath_v, agg_sh, msem, gsem, ssem, zsem):
        cid = lax.axis_index("c")
        s = lax.axis_index("s")
        w = s * NC + cid
        ebase = w * E_PER_W

        with jax.named_scope("sc_zero"):
            zvec = jnp.zeros((16,), jnp.float32)

            def zero_buf(i, carry):
                for j in range(D // 16):
                    gath_v[0, i, pl.ds(j * 16, 16)] = zvec
                return carry

            lax.fori_loop(0, ZR, zero_buf, 0)
            zero_v = gath_v.at[0, pl.ds(0, ZR)]

            for k_ in range(15):
                pltpu.async_copy(
                    zero_v, agg_sh.at[pl.ds((k_ * NS + s) * ZR, ZR)], zsem)

            @pl.when(NS * 15 + s < NZC)
            def _():
                pltpu.sync_copy(
                    zero_v, agg_sh.at[pl.ds((NS * 15 + s) * ZR, ZR)])

            for k_ in range(15):
                pltpu.make_async_copy(
                    zero_v, agg_sh.at[pl.ds(0, ZR)], zsem).wait()
            plsc.subcore_barrier()

        def fire_meta(ci, mb):
            off = ebase + ci * CHUNK
            pltpu.async_copy(nbidx_hbm.at[pl.ds(NNZ + off, CHUNK)],
                             cidx_v.at[mb], msem.at[mb])
            pltpu.async_copy(nbidx_hbm.at[pl.ds(off, CHUNK)],
                             ridx_v.at[mb], msem.at[mb])
            pltpu.async_copy(vals_hbm.at[pl.ds(off, CHUNK)],
                             vals_v.at[mb], msem.at[mb])

        def wait_meta(mb):
            pltpu.make_async_copy(nbidx_hbm.at[pl.ds(0, CHUNK)],
                                  cidx_v.at[mb], msem.at[mb]).wait()
            pltpu.make_async_copy(nbidx_hbm.at[pl.ds(0, CHUNK)],
                                  ridx_v.at[mb], msem.at[mb]).wait()
            pltpu.make_async_copy(vals_hbm.at[pl.ds(0, CHUNK)],
                                  vals_v.at[mb], msem.at[mb]).wait()

        def fire_gather(b, mb):
            pltpu.async_copy(x0_hbm.at[cidx_v.at[mb]], gath_v.at[b],
                             gsem.at[b])

        def wait_gather(b):
            pltpu.make_async_copy(x0_hbm.at[pl.ds(0, CHUNK)], gath_v.at[b],
                                  gsem.at[b]).wait()

        def wait_scatter(b):
            pltpu.make_async_copy(gath_v.at[b],
                                  agg_sh.at[pl.ds(0, CHUNK)],
                                  ssem.at[b]).wait()

        scope_main = jax.named_scope("sc_mainloop")
        scope_main.__enter__()
        for k_ in range(6):
            fire_meta(k_, k_)
        wait_meta(0)
        fire_gather(0, 0)
        wait_meta(1)
        fire_gather(1, 1)

        def turn(c, carry):
            b = lax.rem(c, G)
            mb = lax.rem(c, M)
            wait_gather(b)

            gref = gath_v.at[b]
            vref = vals_v.at[mb]

            @plsc.parallel_loop(0, CHUNK, 1, unroll=16)
            def _(e):
                v16 = plsc.load_gather(vref, [jnp.full((16,), e, jnp.int32)])
                for j in range(D // 16):
                    sl = pl.ds(j * 16, 16)
                    gref[e, sl] = gref[e, sl] * v16

            pltpu.async_copy(gref, agg_sh.at[ridx_v.at[mb]], ssem.at[b],
                             add=True)

            @pl.when(c >= 2)
            def _():
                wait_scatter(lax.rem(c + 2, G))

            @pl.when(c + 6 < NCH)
            def _():
                fire_meta(c + 6, lax.rem(c + 6, M))

            @pl.when(c + 2 < NCH)
            def _():
                wait_meta(lax.rem(c + 2, M))
                fire_gather(lax.rem(c + 2, G), lax.rem(c + 2, M))

            return carry

        lax.fori_loop(0, NCH, turn, 0)
        wait_scatter((NCH - 2) % G)
        wait_scatter((NCH - 1) % G)
        plsc.subcore_barrier()
        scope_main.__exit__(None, None, None)

        with jax.named_scope("sc_writeback"):
            def wb_chunk(idx):
                off = idx * WB
                pltpu.sync_copy(agg_sh.at[pl.ds(off, WB)],
                                out_hbm.at[cid, pl.ds(off, WB)])

            for k_ in range(7):
                wb_chunk(k_ * NS + s)

            @pl.when(NS * 7 + s < NWC)
            def _():
                wb_chunk(NS * 7 + s)

    return k(x0, nbidx, vals)


def _tc_body(p0_ref, p1_ref, w_ref, o_ref):
    acc = p0_ref[...] + p1_ref[...]
    y = jnp.dot(acc, w_ref[...], preferred_element_type=jnp.float32)
    o_ref[...] = jnp.where(y > 0, y, jnp.exp(y) - 1.0)


def _tc_finish(p0, p1, W):
    blk = 400
    grid = N1 // blk
    return pl.pallas_call(
        _tc_body,
        grid=(grid,),
        in_specs=[
            pl.BlockSpec((blk, D), lambda i: (i, 0)),
            pl.BlockSpec((blk, D), lambda i: (i, 0)),
            pl.BlockSpec((D, D), lambda i: (0, 0)),
        ],
        out_specs=pl.BlockSpec((blk, D), lambda i: (i, 0)),
        out_shape=jax.ShapeDtypeStruct((N1, D), jnp.float32),
    )(p0, p1, W)


def kernel(x_0, x_1, nb_indices, nb_values, W):
    nbidx = nb_indices.astype(jnp.int32).reshape(2 * NNZ)
    partials = _sc_segment_sum(x_0, nbidx, nb_values)
    return _tc_finish(partials[0], partials[1], W)


# speedup vs baseline: 3.1157x; 1.1072x over previous
import functools

import jax
import jax.numpy as jnp
from jax import lax
from jax.experimental import pallas as pl
from jax.experimental.pallas import tpu as pltpu
from jax.experimental.pallas import tpu_sc as plsc

N0 = 10000
N1 = 10000
NNZ = 320000
D = 128

NC = 2
NS = 16
NW = NC * NS
E_PER_W = NNZ // NW
CHUNK = 80
NCH = E_PER_W // CHUNK
G = 4
M = 8
ZR = 40
NZC = N1 // ZR
WB = 80
NWC = N1 // WB


def _sc_segment_sum(x0, nbidx, vals):
    mesh = plsc.VectorSubcoreMesh(core_axis_name="c", subcore_axis_name="s")

    @functools.partial(
        pl.kernel,
        out_type=jax.ShapeDtypeStruct((NC, N1, D), jnp.float32),
        mesh=mesh,
        compiler_params=pltpu.CompilerParams(needs_layout_passes=False),
        scratch_types=[
            pltpu.VMEM((M, CHUNK), jnp.int32),
            pltpu.VMEM((M, CHUNK), jnp.int32),
            pltpu.VMEM((M, CHUNK), jnp.float32),
            pltpu.VMEM((G, CHUNK, D), jnp.float32),
            pltpu.VMEM_SHARED((N1, D), jnp.float32),
            pltpu.SemaphoreType.DMA((M,)),
            pltpu.SemaphoreType.DMA((G,)),
            pltpu.SemaphoreType.DMA((G,)),
            pltpu.SemaphoreType.DMA,
        ],
    )
    def k(x0_hbm, nbidx_hbm, vals_hbm, out_hbm,
          cidx_v, ridx_v, vals_v, gath_v, agg_sh, msem, gsem, ssem, zsem):
        cid = lax.axis_index("c")
        s = lax.axis_index("s")
        w = s * NC + cid
        ebase = w * E_PER_W

        with jax.named_scope("sc_zero"):
            zvec = jnp.zeros((16,), jnp.float32)

            def zero_buf(i, carry):
                for j in range(D // 16):
                    gath_v[0, i, pl.ds(j * 16, 16)] = zvec
                return carry

            lax.fori_loop(0, ZR, zero_buf, 0)
            zero_v = gath_v.at[0, pl.ds(0, ZR)]

            for k_ in range(15):
                pltpu.async_copy(
                    zero_v, agg_sh.at[pl.ds((k_ * NS + s) * ZR, ZR)], zsem)

            @pl.when(NS * 15 + s < NZC)
            def _():
                pltpu.sync_copy(
                    zero_v, agg_sh.at[pl.ds((NS * 15 + s) * ZR, ZR)])

            for k_ in range(15):
                pltpu.make_async_copy(
                    zero_v, agg_sh.at[pl.ds(0, ZR)], zsem).wait()
            plsc.subcore_barrier()

        def fire_meta(ci, mb):
            off = ebase + ci * CHUNK
            pltpu.async_copy(nbidx_hbm.at[pl.ds(NNZ + off, CHUNK)],
                             cidx_v.at[mb], msem.at[mb])
            pltpu.async_copy(nbidx_hbm.at[pl.ds(off, CHUNK)],
                             ridx_v.at[mb], msem.at[mb])
            pltpu.async_copy(vals_hbm.at[pl.ds(off, CHUNK)],
                             vals_v.at[mb], msem.at[mb])

        def wait_meta(mb):
            pltpu.make_async_copy(nbidx_hbm.at[pl.ds(0, CHUNK)],
                                  cidx_v.at[mb], msem.at[mb]).wait()
            pltpu.make_async_copy(nbidx_hbm.at[pl.ds(0, CHUNK)],
                                  ridx_v.at[mb], msem.at[mb]).wait()
            pltpu.make_async_copy(vals_hbm.at[pl.ds(0, CHUNK)],
                                  vals_v.at[mb], msem.at[mb]).wait()

        def fire_gather(b, mb):
            pltpu.async_copy(x0_hbm.at[cidx_v.at[mb]], gath_v.at[b],
                             gsem.at[b])

        def wait_gather(b):
            pltpu.make_async_copy(x0_hbm.at[pl.ds(0, CHUNK)], gath_v.at[b],
                                  gsem.at[b]).wait()

        def wait_scatter(b):
            pltpu.make_async_copy(gath_v.at[b],
                                  agg_sh.at[pl.ds(0, CHUNK)],
                                  ssem.at[b]).wait()

        scope_main = jax.named_scope("sc_mainloop")
        scope_main.__enter__()
        for k_ in range(6):
            fire_meta(k_, k_)
        wait_meta(0)
        fire_gather(0, 0)
        wait_meta(1)
        fire_gather(1, 1)

        def turn(c, carry):
            b = lax.rem(c, G)
            mb = lax.rem(c, M)
            wait_gather(b)

            gref = gath_v.at[b]
            vref = vals_v.at[mb]

            @plsc.parallel_loop(0, CHUNK, 1, unroll=16)
            def _(e):
                v16 = plsc.load_gather(vref, [jnp.full((16,), e, jnp.int32)])
                for j in range(D // 16):
                    sl = pl.ds(j * 16, 16)
                    gref[e, sl] = gref[e, sl] * v16

            pltpu.async_copy(gref, agg_sh.at[ridx_v.at[mb]], ssem.at[b],
                             add=True)

            @pl.when(c >= 2)
            def _():
                wait_scatter(lax.rem(c + 2, G))

            @pl.when(c + 6 < NCH)
            def _():
                fire_meta(c + 6, lax.rem(c + 6, M))

            @pl.when(c + 2 < NCH)
            def _():
                wait_meta(lax.rem(c + 2, M))
                fire_gather(lax.rem(c + 2, G), lax.rem(c + 2, M))

            return carry

        lax.fori_loop(0, NCH, turn, 0)
        wait_scatter((NCH - 2) % G)
        wait_scatter((NCH - 1) % G)
        plsc.subcore_barrier()
        scope_main.__exit__(None, None, None)

        with jax.named_scope("sc_writeback"):
            def wb_chunk(idx):
                off = idx * WB
                pltpu.sync_copy(agg_sh.at[pl.ds(off, WB)],
                                out_hbm.at[cid, pl.ds(off, WB)])

            for k_ in range(7):
                wb_chunk(k_ * NS + s)

            @pl.when(NS * 7 + s < NWC)
            def _():
                wb_chunk(NS * 7 + s)

    return k(x0, nbidx, vals)


def _tc_body(p0_ref, p1_ref, w_ref, o_ref):
    acc = p0_ref[0] + p1_ref[0]
    y = jnp.dot(acc, w_ref[...], preferred_element_type=jnp.float32)
    o_ref[...] = jnp.where(y > 0, y, jnp.exp(y) - 1.0)


def _tc_finish(partials, W):
    blk = 2000
    grid = N1 // blk
    return pl.pallas_call(
        _tc_body,
        grid=(grid,),
        in_specs=[
            pl.BlockSpec((1, blk, D), lambda i: (0, i, 0)),
            pl.BlockSpec((1, blk, D), lambda i: (1, i, 0)),
            pl.BlockSpec((D, D), lambda i: (0, 0)),
        ],
        out_specs=pl.BlockSpec((blk, D), lambda i: (i, 0)),
        out_shape=jax.ShapeDtypeStruct((N1, D), jnp.float32),
    )(partials, partials, W)


def kernel(x_0, x_1, nb_indices, nb_values, W):
    nbidx = nb_indices.astype(jnp.int32).reshape(2 * NNZ)
    partials = _sc_segment_sum(x_0, nbidx, nb_values)
    return _tc_finish(partials, W)
